# Initial kernel scaffold; baseline (speedup 1.0000x reference)
import jax
import jax.numpy as jnp
from jax.experimental import pallas as pl


def kernel(nf, edge_index, ef, W_node, W_edge, attn_l, attn_r, attn_e, edge_weights, bias):
    raise NotImplementedError("write your pallas kernel here")


# speedup vs baseline: 4.1512x; 4.1512x over previous
import functools
import jax
import jax.numpy as jnp
from jax import lax
from jax.experimental import pallas as pl
from jax.experimental.pallas import tpu as pltpu
from jax.experimental.pallas import tpu_sc as plsc

NN = 10000
NE = 320000
FF = 128
WW = 144
NC = 2
NS = 16
NWK = NC * NS
EPW = NE // NWK
CB = 80
NPH = 5120
NTR = 128
TBL = NPH + NTR
RPS = TBL // NS
ZCH = 8
BE = 3200


def _k1_body(nf_ref, waug_ref, attn2_ref, wedge_ref, ew_ref, attne_ref,
             zaug_ref, alr_ref, wbaug_ref):
    z = jnp.dot(nf_ref[...], waug_ref[...].T, preferred_element_type=jnp.float32)
    ones = jnp.ones((z.shape[0], 1), jnp.float32)
    zaug_ref[:, :FF] = z[:, :FF]
    zaug_ref[:, FF:FF + 1] = ones
    zaug_ref[:, FF + 1:] = jnp.zeros((z.shape[0], WW - FF - 1), jnp.float32)
    zc = z[:, :FF]
    al = jnp.sum(zc * attn2_ref[0:1, :], axis=1)
    ar = jnp.sum(zc * attn2_ref[1:2, :], axis=1)
    alr_ref[0:1, :] = al[None, :]
    alr_ref[1:2, :] = ar[None, :]
    alr_ref[2:, :] = jnp.zeros((6, z.shape[0]), jnp.float32)
    q = jnp.dot(attne_ref[0:1, :], ew_ref[...], preferred_element_type=jnp.float32)
    we = jnp.dot(q, wedge_ref[...], preferred_element_type=jnp.float32)
    wbaug_ref[:FF, :] = wedge_ref[...]
    wbaug_ref[FF:FF + 1, :] = we
    wbaug_ref[FF + 1:, :] = jnp.zeros((WW - FF - 1, 16), jnp.float32)


def _k2_body(ef_ref, wbaug_ref, cb_ref, zeb_ref):
    zeb_ref[...] = jnp.dot(ef_ref[...], wbaug_ref[...].T,
                           preferred_element_type=jnp.float32) + cb_ref[...]


def _k4_body(hraw_ref, out_ref):
    p_lo = hraw_ref[0, 0, :NPH, :] + hraw_ref[1, 0, :NPH, :]
    p_hi = hraw_ref[0, 1, :NN - NPH, :] + hraw_ref[1, 1, :NN - NPH, :]
    p = jnp.concatenate([p_lo, p_hi], axis=0)
    d = p[:, FF:FF + 1]
    safe = jnp.where(d > 0.0, d, 1.0)
    out_ref[...] = jnp.where(d > 0.0, p[:, :FF] / safe, 0.0)


def _k3_body(zaug_hbm, alr_hbm, src_hbm, dst_hbm, zeb_hbm,
             fused_hbm, hraw_hbm, exs_hbm,
             al_t, ar_t, src_t, dst_t, dstx_t, ex_t, rows_t, zeb_t, fbuf, zbuf,
             hacc, sem1, sem2):
    c = lax.axis_index("c")
    s = lax.axis_index("s")
    wid = c * NS + s
    ebase = wid * EPW

    lane = jnp.arange(16, dtype=jnp.int32)
    col_ee = jnp.full((16,), FF, jnp.int32)

    def _zero_own_slice():
        @pl.loop(0, RPS, step=ZCH)
        def _zero_hacc(r):
            pltpu.sync_copy(zbuf, hacc.at[pl.ds(s * RPS + r, ZCH)])

    @pl.loop(0, ZCH)
    def _zero_zbuf(i):
        for k in range(WW // 16):
            zbuf[i, pl.ds(k * 16, 16)] = jnp.zeros((16,), jnp.float32)

    _zero_own_slice()

    pltpu.sync_copy(alr_hbm.at[0], al_t)
    pltpu.sync_copy(alr_hbm.at[1], ar_t)
    plsc.subcore_barrier()

    @pl.loop(0, EPW, step=CB)
    def _chunk(off):
        base = ebase + off
        pltpu.sync_copy(src_hbm.at[pl.ds(base, CB)], src_t)
        pltpu.sync_copy(dst_hbm.at[pl.ds(base, CB)], dst_t)
        grow = pltpu.async_copy(zaug_hbm.at[src_t], rows_t, sem1)
        gzeb = pltpu.async_copy(zeb_hbm.at[pl.ds(base, CB)], zeb_t, sem2)
        gzeb.wait()

        for g in range(CB // 16):
            sl = pl.ds(g * 16, 16)
            si = src_t[sl]
            di = dst_t[sl]
            av = plsc.load_gather(al_t, [si])
            bv = plsc.load_gather(ar_t, [di])
            ev = plsc.load_gather(zeb_t, [lane + g * 16, col_ee])
            x = av + bv + ev
            x = jnp.where(x >= 0.0, x, x * 0.01)
            ex_t[sl] = jnp.exp(x)
            trash = NPH + ((lane + g * 16) & (NTR - 1))
            dstx_t[sl] = jnp.where(di < NPH, di, trash)

        grow.wait()

        @pl.loop(0, CB)
        def _edge(i):
            exs = ex_t[pl.ds(i, 16)][0]
            for k in range(WW // 16):
                ksl = pl.ds(k * 16, 16)
                r = rows_t[i, ksl]
                if k < FF // 16:
                    fbuf[i, ksl] = r + zeb_t[i, ksl]
                rows_t[i, ksl] = r * exs

        pltpu.sync_copy(fbuf, fused_hbm.at[pl.ds(base, CB)])
        pltpu.sync_copy(ex_t.at[pl.ds(0, CB)], exs_hbm.at[pl.ds(base, CB)])
        pltpu.sync_copy(rows_t, hacc.at[dstx_t], add=True)

    plsc.subcore_barrier()
    pltpu.sync_copy(hacc.at[pl.ds(s * RPS, RPS)],
                    hraw_hbm.at[c, 0, pl.ds(s * RPS, RPS)])
    _zero_own_slice()
    plsc.subcore_barrier()

    @pl.loop(0, EPW, step=CB)
    def _chunk2(off):
        base = ebase + off
        pltpu.sync_copy(src_hbm.at[pl.ds(base, CB)], src_t)
        pltpu.sync_copy(dst_hbm.at[pl.ds(base, CB)], dst_t)
        grow = pltpu.async_copy(zaug_hbm.at[src_t], rows_t, sem1)
        pltpu.sync_copy(exs_hbm.at[pl.ds(base, CB)], ex_t.at[pl.ds(0, CB)])

        for g in range(CB // 16):
            sl = pl.ds(g * 16, 16)
            di = dst_t[sl]
            trash = NPH + ((lane + g * 16) & (NTR - 1))
            dstx_t[sl] = jnp.where(di >= NPH, di - NPH, trash)

        grow.wait()

        @pl.loop(0, CB)
        def _edge2(i):
            exs = ex_t[pl.ds(i, 16)][0]
            for k in range(WW // 16):
                ksl = pl.ds(k * 16, 16)
                rows_t[i, ksl] = rows_t[i, ksl] * exs

        pltpu.sync_copy(rows_t, hacc.at[dstx_t], add=True)

    plsc.subcore_barrier()
    pltpu.sync_copy(hacc.at[pl.ds(s * RPS, RPS)],
                    hraw_hbm.at[c, 1, pl.ds(s * RPS, RPS)])


@functools.cache
def _edge_sc():
  return pl.kernel(
    _k3_body,
    out_type=[jax.ShapeDtypeStruct((NE, FF), jnp.float32),
              jax.ShapeDtypeStruct((NC, 2, TBL, WW), jnp.float32),
              jax.ShapeDtypeStruct((NE,), jnp.float32)],
    mesh=plsc.VectorSubcoreMesh(core_axis_name="c", subcore_axis_name="s",
                                num_cores=NC, num_subcores=NS),
    compiler_params=pltpu.CompilerParams(needs_layout_passes=False,
                                         use_tc_tiling_on_sc=False),
    scratch_types=[
        pltpu.VMEM((NN,), jnp.float32),
        pltpu.VMEM((NN,), jnp.float32),
        pltpu.VMEM((CB,), jnp.int32),
        pltpu.VMEM((CB,), jnp.int32),
        pltpu.VMEM((CB,), jnp.int32),
        pltpu.VMEM((CB + 16,), jnp.float32),
        pltpu.VMEM((CB, WW), jnp.float32),
        pltpu.VMEM((CB, WW), jnp.float32),
        pltpu.VMEM((CB, FF), jnp.float32),
        pltpu.VMEM((ZCH, WW), jnp.float32),
        pltpu.VMEM_SHARED((TBL, WW), jnp.float32),
        pltpu.SemaphoreType.DMA,
        pltpu.SemaphoreType.DMA,
    ],
  )


@jax.jit
def _run(nf, src, dst, ef, W_node, W_edge, attn_l, attn_r, attn_e, edge_weights, bias):
    waug = jnp.concatenate([W_node, jnp.zeros((WW - FF, W_node.shape[1]), jnp.float32)], axis=0)
    attn2 = jnp.concatenate([attn_l, attn_r, jnp.zeros((6, FF), jnp.float32)], axis=0)
    attne8 = jnp.concatenate([attn_e, jnp.zeros((7, FF), jnp.float32)], axis=0)
    cbv = jnp.concatenate([bias, jnp.zeros((WW - FF,), jnp.float32)])[None, :]

    zaug, alr, wbaug = pl.pallas_call(
        _k1_body,
        out_shape=[jax.ShapeDtypeStruct((NN, WW), jnp.float32),
                   jax.ShapeDtypeStruct((8, NN), jnp.float32),
                   jax.ShapeDtypeStruct((WW, 16), jnp.float32)],
    )(nf, waug, attn2, W_edge, edge_weights, attne8)

    zeb = pl.pallas_call(
        _k2_body,
        grid=(NE // BE,),
        in_specs=[pl.BlockSpec((BE, 16), lambda i: (i, 0)),
                  pl.BlockSpec((WW, 16), lambda i: (0, 0)),
                  pl.BlockSpec((1, WW), lambda i: (0, 0))],
        out_specs=pl.BlockSpec((BE, WW), lambda i: (i, 0)),
        out_shape=jax.ShapeDtypeStruct((NE, WW), jnp.float32),
    )(ef, wbaug, cbv)

    fused, hraw, _ = _edge_sc()(zaug, alr, src, dst, zeb)

    n_out = pl.pallas_call(
        _k4_body,
        out_shape=jax.ShapeDtypeStruct((NN, FF), jnp.float32),
    )(hraw)
    return n_out, fused


def kernel(nf, edge_index, ef, W_node, W_edge, attn_l, attn_r, attn_e, edge_weights, bias):
    src = edge_index[0]
    dst = edge_index[1]
    n_out, fused = _run(nf, src, dst, ef.reshape(-1, 16), W_node, W_edge,
                        attn_l, attn_r, attn_e, edge_weights, bias)
    return n_out, fused.reshape(NE, 1, FF)


# speedup vs baseline: 4.3947x; 1.0587x over previous
import functools
import jax
import jax.numpy as jnp
from jax import lax
from jax.experimental import pallas as pl
from jax.experimental.pallas import tpu as pltpu
from jax.experimental.pallas import tpu_sc as plsc

NN = 10000
NE = 320000
FF = 128
WW = 144
NC = 2
NS = 16
NWK = NC * NS
EPW = NE // NWK
CB = 80
NPH = 5120
NTR = 128
TBL = NPH + NTR
RPS = TBL // NS
ZCH = 8
BE = 3200


def _k1_body(nf_ref, waug_ref, attn2_ref, wedge_ref, ew_ref, attne_ref,
             zaug_ref, alr_ref, wbaug_ref):
    z = jnp.dot(nf_ref[...], waug_ref[...].T, preferred_element_type=jnp.float32)
    ones = jnp.ones((z.shape[0], 1), jnp.float32)
    zaug_ref[:, :FF] = z[:, :FF]
    zaug_ref[:, FF:FF + 1] = ones
    zaug_ref[:, FF + 1:] = jnp.zeros((z.shape[0], WW - FF - 1), jnp.float32)
    zc = z[:, :FF]
    al = jnp.sum(zc * attn2_ref[0:1, :], axis=1)
    ar = jnp.sum(zc * attn2_ref[1:2, :], axis=1)
    alr_ref[0:1, :] = al[None, :]
    alr_ref[1:2, :] = ar[None, :]
    alr_ref[2:, :] = jnp.zeros((6, z.shape[0]), jnp.float32)
    q = jnp.dot(attne_ref[0:1, :], ew_ref[...], preferred_element_type=jnp.float32)
    we = jnp.dot(q, wedge_ref[...], preferred_element_type=jnp.float32)
    wbaug_ref[:FF, :] = wedge_ref[...]
    wbaug_ref[FF:FF + 1, :] = we
    wbaug_ref[FF + 1:, :] = jnp.zeros((WW - FF - 1, 16), jnp.float32)


def _k2_body(ef_ref, wbaug_ref, cb_ref, zeb_ref):
    zeb_ref[...] = jnp.dot(ef_ref[...], wbaug_ref[...].T,
                           preferred_element_type=jnp.float32) + cb_ref[...]


def _k4_body(hraw_ref, out_ref):
    p_lo = hraw_ref[0, 0, :NPH, :] + hraw_ref[1, 0, :NPH, :]
    p_hi = hraw_ref[0, 1, :NN - NPH, :] + hraw_ref[1, 1, :NN - NPH, :]
    p = jnp.concatenate([p_lo, p_hi], axis=0)
    d = p[:, FF:FF + 1]
    safe = jnp.where(d > 0.0, d, 1.0)
    out_ref[...] = jnp.where(d > 0.0, p[:, :FF] / safe, 0.0)


def _k3_body(zaug_hbm, alr_hbm, src_hbm, dst_hbm, zeb_hbm,
             fused_hbm, hraw_hbm, exs_hbm,
             al_t, ar_t,
             srcA, dstA, rowsA, srcB, dstB, rowsB,
             dstx_t, ex_t, zeb_t, fbuf,
             zbuf, hacc,
             semrA, semrB):
    c = lax.axis_index("c")
    s = lax.axis_index("s")
    wid = c * NS + s
    ebase = wid * EPW

    lane = jnp.arange(16, dtype=jnp.int32)
    col_ee = jnp.full((16,), FF, jnp.int32)

    A = (srcA, dstA, rowsA, semrA)
    B = (srcB, dstB, rowsB, semrB)

    def _zero_own_slice():
        @pl.loop(0, RPS, step=ZCH)
        def _zero_hacc(r):
            pltpu.sync_copy(zbuf, hacc.at[pl.ds(s * RPS + r, ZCH)])

    @pl.loop(0, ZCH)
    def _zero_zbuf(i):
        for k in range(WW // 16):
            zbuf[i, pl.ds(k * 16, 16)] = jnp.zeros((16,), jnp.float32)

    _zero_own_slice()

    pltpu.sync_copy(alr_hbm.at[0], al_t)
    pltpu.sync_copy(alr_hbm.at[1], ar_t)
    plsc.subcore_barrier()

    def issue_idx(off, S):
        src_t, dst_t, semr = S[0], S[1], S[3]
        base = ebase + off
        pltpu.async_copy(src_hbm.at[pl.ds(base, CB)], src_t, semr)
        pltpu.async_copy(dst_hbm.at[pl.ds(base, CB)], dst_t, semr)

    def wait_idx(off, S):
        src_t, dst_t, semr = S[0], S[1], S[3]
        base = ebase + off
        pltpu.make_async_copy(src_hbm.at[pl.ds(base, CB)], src_t, semr).wait()
        pltpu.make_async_copy(dst_hbm.at[pl.ds(base, CB)], dst_t, semr).wait()

    def issue_big(off, S, phase1):
        src_t, rows_t, semr = S[0], S[2], S[3]
        pltpu.async_copy(zaug_hbm.at[src_t], rows_t, semr)

    def compute1(off, S, prefetch):
        src_t, dst_t, rows_t, semr = S
        base = ebase + off
        pltpu.sync_copy(zeb_hbm.at[pl.ds(base, CB)], zeb_t)

        for g in range(CB // 16):
            sl = pl.ds(g * 16, 16)
            si = src_t[sl]
            di = dst_t[sl]
            av = plsc.load_gather(al_t, [si])
            bv = plsc.load_gather(ar_t, [di])
            ev = plsc.load_gather(zeb_t, [lane + g * 16, col_ee])
            x = av + bv + ev
            x = jnp.where(x >= 0.0, x, x * 0.01)
            ex_t[sl] = jnp.exp(x)
            trash = NPH + ((lane + g * 16) & (NTR - 1))
            dstx_t[sl] = jnp.where(di < NPH, di, trash)

        pltpu.make_async_copy(zaug_hbm.at[src_t], rows_t, semr).wait()
        prefetch()

        @pl.loop(0, CB)
        def _edge(i):
            exs = ex_t[pl.ds(i, 16)][0]
            for k in range(WW // 16):
                ksl = pl.ds(k * 16, 16)
                r = rows_t[i, ksl]
                if k < FF // 16:
                    fbuf[i, ksl] = r + zeb_t[i, ksl]
                rows_t[i, ksl] = r * exs

        pltpu.sync_copy(fbuf, fused_hbm.at[pl.ds(base, CB)])
        pltpu.sync_copy(ex_t.at[pl.ds(0, CB)], exs_hbm.at[pl.ds(base, CB)])
        pltpu.sync_copy(rows_t, hacc.at[dstx_t], add=True)

    issue_idx(0, A)
    wait_idx(0, A)
    issue_big(0, A, True)
    issue_idx(CB, B)

    @pl.loop(0, (EPW // CB - 1) // 2 * 2 * CB, step=2 * CB)
    def _pair(off):
        wait_idx(off + CB, B)
        issue_big(off + CB, B, True)

        def _pfA():
            issue_idx(off + 2 * CB, A)
        compute1(off, A, _pfA)

        wait_idx(off + 2 * CB, A)
        issue_big(off + 2 * CB, A, True)

        def _pfB():
            @pl.when(off + 3 * CB < EPW)
            def _():
                issue_idx(off + 3 * CB, B)
        compute1(off + CB, B, _pfB)

    compute1(EPW - CB, A, lambda: None)

    plsc.subcore_barrier()
    pltpu.sync_copy(hacc.at[pl.ds(s * RPS, RPS)],
                    hraw_hbm.at[c, 0, pl.ds(s * RPS, RPS)])
    _zero_own_slice()
    plsc.subcore_barrier()

    @pl.loop(0, EPW, step=CB)
    def _chunk2(off):
        base = ebase + off
        pltpu.sync_copy(src_hbm.at[pl.ds(base, CB)], srcA)
        pltpu.sync_copy(dst_hbm.at[pl.ds(base, CB)], dstA)
        grow = pltpu.async_copy(zaug_hbm.at[srcA], rowsA, semrA)
        pltpu.sync_copy(exs_hbm.at[pl.ds(base, CB)], ex_t.at[pl.ds(0, CB)])

        for g in range(CB // 16):
            sl = pl.ds(g * 16, 16)
            di = dstA[sl]
            trash = NPH + ((lane + g * 16) & (NTR - 1))
            dstx_t[sl] = jnp.where(di >= NPH, di - NPH, trash)

        grow.wait()

        @pl.loop(0, CB)
        def _edge2(i):
            exs = ex_t[pl.ds(i, 16)][0]
            for k in range(WW // 16):
                ksl = pl.ds(k * 16, 16)
                rowsA[i, ksl] = rowsA[i, ksl] * exs

        pltpu.sync_copy(rowsA, hacc.at[dstx_t], add=True)

    plsc.subcore_barrier()
    pltpu.sync_copy(hacc.at[pl.ds(s * RPS, RPS)],
                    hraw_hbm.at[c, 1, pl.ds(s * RPS, RPS)])


@functools.cache
def _edge_sc():
  return pl.kernel(
    _k3_body,
    out_type=[jax.ShapeDtypeStruct((NE, FF), jnp.float32),
              jax.ShapeDtypeStruct((NC, 2, TBL, WW), jnp.float32),
              jax.ShapeDtypeStruct((NE,), jnp.float32)],
    mesh=plsc.VectorSubcoreMesh(core_axis_name="c", subcore_axis_name="s",
                                num_cores=NC, num_subcores=NS),
    compiler_params=pltpu.CompilerParams(needs_layout_passes=False,
                                         use_tc_tiling_on_sc=False),
    scratch_types=(
        [pltpu.VMEM((NN,), jnp.float32),
         pltpu.VMEM((NN,), jnp.float32)]
        + 2 * [pltpu.VMEM((CB,), jnp.int32),
               pltpu.VMEM((CB,), jnp.int32),
               pltpu.VMEM((CB, WW), jnp.float32)]
        + [pltpu.VMEM((CB,), jnp.int32),
           pltpu.VMEM((CB + 16,), jnp.float32),
           pltpu.VMEM((CB, WW), jnp.float32),
           pltpu.VMEM((CB, FF), jnp.float32),
           pltpu.VMEM((ZCH, WW), jnp.float32),
           pltpu.VMEM_SHARED((TBL, WW), jnp.float32)]
        + 2 * [pltpu.SemaphoreType.DMA]
    ),
  )


@jax.jit
def _run(nf, src, dst, ef, W_node, W_edge, attn_l, attn_r, attn_e, edge_weights, bias):
    waug = jnp.concatenate([W_node, jnp.zeros((WW - FF, W_node.shape[1]), jnp.float32)], axis=0)
    attn2 = jnp.concatenate([attn_l, attn_r, jnp.zeros((6, FF), jnp.float32)], axis=0)
    attne8 = jnp.concatenate([attn_e, jnp.zeros((7, FF), jnp.float32)], axis=0)
    cbv = jnp.concatenate([bias, jnp.zeros((WW - FF,), jnp.float32)])[None, :]

    zaug, alr, wbaug = pl.pallas_call(
        _k1_body,
        out_shape=[jax.ShapeDtypeStruct((NN, WW), jnp.float32),
                   jax.ShapeDtypeStruct((8, NN), jnp.float32),
                   jax.ShapeDtypeStruct((WW, 16), jnp.float32)],
    )(nf, waug, attn2, W_edge, edge_weights, attne8)

    zeb = pl.pallas_call(
        _k2_body,
        grid=(NE // BE,),
        in_specs=[pl.BlockSpec((BE, 16), lambda i: (i, 0)),
                  pl.BlockSpec((WW, 16), lambda i: (0, 0)),
                  pl.BlockSpec((1, WW), lambda i: (0, 0))],
        out_specs=pl.BlockSpec((BE, WW), lambda i: (i, 0)),
        out_shape=jax.ShapeDtypeStruct((NE, WW), jnp.float32),
    )(ef, wbaug, cbv)

    fused, hraw, _ = _edge_sc()(zaug, alr, src, dst, zeb)

    n_out = pl.pallas_call(
        _k4_body,
        out_shape=jax.ShapeDtypeStruct((NN, FF), jnp.float32),
    )(hraw)
    return n_out, fused


def kernel(nf, edge_index, ef, W_node, W_edge, attn_l, attn_r, attn_e, edge_weights, bias):
    src = edge_index[0]
    dst = edge_index[1]
    n_out, fused = _run(nf, src, dst, ef.reshape(-1, 16), W_node, W_edge,
                        attn_l, attn_r, attn_e, edge_weights, bias)
    return n_out, fused.reshape(NE, 1, FF)


# speedup vs baseline: 4.8940x; 1.1136x over previous
import functools
import jax
import jax.numpy as jnp
from jax import lax
from jax.experimental import pallas as pl
from jax.experimental.pallas import tpu as pltpu
from jax.experimental.pallas import tpu_sc as plsc

NN = 10000
NE = 320000
FF = 128
WW = 144
NC = 2
NS = 16
NWK = NC * NS
EPW = NE // NWK
CB = 80
NPH = 5120
NTR = 128
TBL = NPH + NTR
RPS = TBL // NS
ZCH = 8
BE = 3200


def _k1_body(nf_ref, waug_ref, attn2_ref, wedge_ref, ew_ref, attne_ref,
             zaug_ref, alr_ref, wbaug_ref):
    z = jnp.dot(nf_ref[...], waug_ref[...].T, preferred_element_type=jnp.float32)
    ones = jnp.ones((z.shape[0], 1), jnp.float32)
    zaug_ref[:, :FF] = z[:, :FF]
    zaug_ref[:, FF:FF + 1] = ones
    zaug_ref[:, FF + 1:] = jnp.zeros((z.shape[0], WW - FF - 1), jnp.float32)
    zc = z[:, :FF]
    al = jnp.sum(zc * attn2_ref[0:1, :], axis=1)
    ar = jnp.sum(zc * attn2_ref[1:2, :], axis=1)
    alr_ref[0:1, :] = al[None, :]
    alr_ref[1:2, :] = ar[None, :]
    alr_ref[2:, :] = jnp.zeros((6, z.shape[0]), jnp.float32)
    q = jnp.dot(attne_ref[0:1, :], ew_ref[...], preferred_element_type=jnp.float32)
    we = jnp.dot(q, wedge_ref[...], preferred_element_type=jnp.float32)
    wbaug_ref[:FF, :] = wedge_ref[...]
    wbaug_ref[FF:FF + 1, :] = we
    wbaug_ref[FF + 1:, :] = jnp.zeros((WW - FF - 1, 16), jnp.float32)


def _k2_body(ef_ref, wbaug_ref, cb_ref, zeb_ref):
    zeb_ref[...] = jnp.dot(ef_ref[...], wbaug_ref[...].T,
                           preferred_element_type=jnp.float32) + cb_ref[...]


def _k4_body(hraw_ref, out_ref):
    p_lo = hraw_ref[0, 0, :NPH, :] + hraw_ref[1, 0, :NPH, :]
    p_hi = hraw_ref[0, 1, :NN - NPH, :] + hraw_ref[1, 1, :NN - NPH, :]
    p = jnp.concatenate([p_lo, p_hi], axis=0)
    d = p[:, FF:FF + 1]
    safe = jnp.where(d > 0.0, d, 1.0)
    out_ref[...] = jnp.where(d > 0.0, p[:, :FF] / safe, 0.0)


def _k3_body(zaug_hbm, alr_hbm, src_hbm, dst_hbm, zeb_hbm,
             fused_hbm, hraw_hbm, exs_hbm,
             al_t, ar_t,
             srcA, dstA, rowsA, srcB, dstB, rowsB,
             dstx_t, ex_t, zeb_t, fbuf,
             zbuf, hacc,
             semrA, semrB):
    c = lax.axis_index("c")
    s = lax.axis_index("s")
    wid = c * NS + s
    ebase = wid * EPW

    lane = jnp.arange(16, dtype=jnp.int32)
    col_ee = jnp.full((16,), FF, jnp.int32)

    A = (srcA, dstA, rowsA, semrA)
    B = (srcB, dstB, rowsB, semrB)

    def _zero_own_slice():
        @pl.loop(0, RPS, step=ZCH)
        def _zero_hacc(r):
            pltpu.sync_copy(zbuf, hacc.at[pl.ds(s * RPS + r, ZCH)])

    @pl.loop(0, ZCH)
    def _zero_zbuf(i):
        for k in range(WW // 16):
            zbuf[i, pl.ds(k * 16, 16)] = jnp.zeros((16,), jnp.float32)

    _zero_own_slice()

    pltpu.sync_copy(alr_hbm.at[0], al_t)
    pltpu.sync_copy(alr_hbm.at[1], ar_t)
    plsc.subcore_barrier()

    def issue_idx(off, S):
        src_t, dst_t, semr = S[0], S[1], S[3]
        base = ebase + off
        pltpu.async_copy(src_hbm.at[pl.ds(base, CB)], src_t, semr)
        pltpu.async_copy(dst_hbm.at[pl.ds(base, CB)], dst_t, semr)

    def wait_idx(off, S):
        src_t, dst_t, semr = S[0], S[1], S[3]
        base = ebase + off
        pltpu.make_async_copy(src_hbm.at[pl.ds(base, CB)], src_t, semr).wait()
        pltpu.make_async_copy(dst_hbm.at[pl.ds(base, CB)], dst_t, semr).wait()

    def issue_big(off, S, phase1):
        src_t, rows_t, semr = S[0], S[2], S[3]
        pltpu.async_copy(zaug_hbm.at[src_t], rows_t, semr)

    def compute1(off, S, prefetch):
        src_t, dst_t, rows_t, semr = S
        base = ebase + off
        pltpu.sync_copy(zeb_hbm.at[pl.ds(base, CB)], zeb_t)

        for g in range(CB // 16):
            sl = pl.ds(g * 16, 16)
            si = src_t[sl]
            di = dst_t[sl]
            av = plsc.load_gather(al_t, [si])
            bv = plsc.load_gather(ar_t, [di])
            ev = plsc.load_gather(zeb_t, [lane + g * 16, col_ee])
            x = av + bv + ev
            x = jnp.where(x >= 0.0, x, x * 0.01)
            ex_t[sl] = jnp.exp(x)
            trash = NPH + ((lane + g * 16) & (NTR - 1))
            dstx_t[sl] = jnp.where(di < NPH, di, trash)

        pltpu.make_async_copy(zaug_hbm.at[src_t], rows_t, semr).wait()
        prefetch()

        @pl.loop(0, CB)
        def _edge(i):
            exs = ex_t[pl.ds(i, 16)][0]
            for k in range(WW // 16):
                ksl = pl.ds(k * 16, 16)
                r = rows_t[i, ksl]
                if k < FF // 16:
                    fbuf[i, ksl] = r + zeb_t[i, ksl]
                rows_t[i, ksl] = r * exs

        pltpu.sync_copy(fbuf, fused_hbm.at[pl.ds(base, CB)])
        pltpu.sync_copy(ex_t.at[pl.ds(0, CB)], exs_hbm.at[pl.ds(base, CB)])
        pltpu.sync_copy(rows_t, hacc.at[dstx_t], add=True)

    issue_idx(0, A)
    wait_idx(0, A)
    issue_big(0, A, True)
    issue_idx(CB, B)

    @pl.loop(0, (EPW // CB - 1) // 2 * 2 * CB, step=2 * CB)
    def _pair(off):
        wait_idx(off + CB, B)
        issue_big(off + CB, B, True)

        def _pfA():
            issue_idx(off + 2 * CB, A)
        compute1(off, A, _pfA)

        wait_idx(off + 2 * CB, A)
        issue_big(off + 2 * CB, A, True)

        def _pfB():
            @pl.when(off + 3 * CB < EPW)
            def _():
                issue_idx(off + 3 * CB, B)
        compute1(off + CB, B, _pfB)

    compute1(EPW - CB, A, lambda: None)

    plsc.subcore_barrier()
    pltpu.sync_copy(hacc.at[pl.ds(s * RPS, RPS)],
                    hraw_hbm.at[c, 0, pl.ds(s * RPS, RPS)])
    _zero_own_slice()
    plsc.subcore_barrier()

    def compute2(off, S, prefetch):
        src_t, dst_t, rows_t, semr = S
        base = ebase + off
        pltpu.sync_copy(exs_hbm.at[pl.ds(base, CB)], ex_t.at[pl.ds(0, CB)])

        for g in range(CB // 16):
            sl = pl.ds(g * 16, 16)
            di = dst_t[sl]
            trash = NPH + ((lane + g * 16) & (NTR - 1))
            dstx_t[sl] = jnp.where(di >= NPH, di - NPH, trash)

        pltpu.make_async_copy(zaug_hbm.at[src_t], rows_t, semr).wait()
        prefetch()

        @pl.loop(0, CB)
        def _edge2(i):
            exs = ex_t[pl.ds(i, 16)][0]
            for k in range(WW // 16):
                ksl = pl.ds(k * 16, 16)
                rows_t[i, ksl] = rows_t[i, ksl] * exs

        pltpu.sync_copy(rows_t, hacc.at[dstx_t], add=True)

    issue_idx(0, A)
    wait_idx(0, A)
    issue_big(0, A, False)
    issue_idx(CB, B)

    @pl.loop(0, (EPW // CB - 1) // 2 * 2 * CB, step=2 * CB)
    def _pair2(off):
        wait_idx(off + CB, B)
        issue_big(off + CB, B, False)

        def _pfA():
            issue_idx(off + 2 * CB, A)
        compute2(off, A, _pfA)

        wait_idx(off + 2 * CB, A)
        issue_big(off + 2 * CB, A, False)

        def _pfB():
            @pl.when(off + 3 * CB < EPW)
            def _():
                issue_idx(off + 3 * CB, B)
        compute2(off + CB, B, _pfB)

    compute2(EPW - CB, A, lambda: None)

    plsc.subcore_barrier()
    pltpu.sync_copy(hacc.at[pl.ds(s * RPS, RPS)],
                    hraw_hbm.at[c, 1, pl.ds(s * RPS, RPS)])


@functools.cache
def _edge_sc():
  return pl.kernel(
    _k3_body,
    out_type=[jax.ShapeDtypeStruct((NE, FF), jnp.float32),
              jax.ShapeDtypeStruct((NC, 2, TBL, WW), jnp.float32),
              jax.ShapeDtypeStruct((NE,), jnp.float32)],
    mesh=plsc.VectorSubcoreMesh(core_axis_name="c", subcore_axis_name="s",
                                num_cores=NC, num_subcores=NS),
    compiler_params=pltpu.CompilerParams(needs_layout_passes=False,
                                         use_tc_tiling_on_sc=False),
    scratch_types=(
        [pltpu.VMEM((NN,), jnp.float32),
         pltpu.VMEM((NN,), jnp.float32)]
        + 2 * [pltpu.VMEM((CB,), jnp.int32),
               pltpu.VMEM((CB,), jnp.int32),
               pltpu.VMEM((CB, WW), jnp.float32)]
        + [pltpu.VMEM((CB,), jnp.int32),
           pltpu.VMEM((CB + 16,), jnp.float32),
           pltpu.VMEM((CB, WW), jnp.float32),
           pltpu.VMEM((CB, FF), jnp.float32),
           pltpu.VMEM((ZCH, WW), jnp.float32),
           pltpu.VMEM_SHARED((TBL, WW), jnp.float32)]
        + 2 * [pltpu.SemaphoreType.DMA]
    ),
  )


@jax.jit
def _run(nf, src, dst, ef, W_node, W_edge, attn_l, attn_r, attn_e, edge_weights, bias):
    waug = jnp.concatenate([W_node, jnp.zeros((WW - FF, W_node.shape[1]), jnp.float32)], axis=0)
    attn2 = jnp.concatenate([attn_l, attn_r, jnp.zeros((6, FF), jnp.float32)], axis=0)
    attne8 = jnp.concatenate([attn_e, jnp.zeros((7, FF), jnp.float32)], axis=0)
    cbv = jnp.concatenate([bias, jnp.zeros((WW - FF,), jnp.float32)])[None, :]

    zaug, alr, wbaug = pl.pallas_call(
        _k1_body,
        out_shape=[jax.ShapeDtypeStruct((NN, WW), jnp.float32),
                   jax.ShapeDtypeStruct((8, NN), jnp.float32),
                   jax.ShapeDtypeStruct((WW, 16), jnp.float32)],
    )(nf, waug, attn2, W_edge, edge_weights, attne8)

    zeb = pl.pallas_call(
        _k2_body,
        grid=(NE // BE,),
        in_specs=[pl.BlockSpec((BE, 16), lambda i: (i, 0)),
                  pl.BlockSpec((WW, 16), lambda i: (0, 0)),
                  pl.BlockSpec((1, WW), lambda i: (0, 0))],
        out_specs=pl.BlockSpec((BE, WW), lambda i: (i, 0)),
        out_shape=jax.ShapeDtypeStruct((NE, WW), jnp.float32),
    )(ef, wbaug, cbv)

    fused, hraw, _ = _edge_sc()(zaug, alr, src, dst, zeb)

    n_out = pl.pallas_call(
        _k4_body,
        out_shape=jax.ShapeDtypeStruct((NN, FF), jnp.float32),
    )(hraw)
    return n_out, fused


def kernel(nf, edge_index, ef, W_node, W_edge, attn_l, attn_r, attn_e, edge_weights, bias):
    src = edge_index[0]
    dst = edge_index[1]
    n_out, fused = _run(nf, src, dst, ef.reshape(-1, 16), W_node, W_edge,
                        attn_l, attn_r, attn_e, edge_weights, bias)
    return n_out, fused.reshape(NE, 1, FF)


# speedup vs baseline: 5.6849x; 1.1616x over previous
import functools
import jax
import jax.numpy as jnp
from jax import lax
from jax.experimental import pallas as pl
from jax.experimental.pallas import tpu as pltpu
from jax.experimental.pallas import tpu_sc as plsc

NN = 10000
NE = 320000
FF = 128
WW = 144
NC = 2
NS = 16
NWK = NC * NS
EPW = NE // NWK
CB = 80
NPH = 5120
NTR = 128
TBL = NPH + NTR
RPS = TBL // NS
ZCH = 8
BE = 3200


def _k1_body(nf_ref, waug_ref, attn2_ref, wedge_ref, ew_ref, attne_ref,
             zaug_ref, alr_ref, wbaug_ref):
    z = jnp.dot(nf_ref[...], waug_ref[...].T, preferred_element_type=jnp.float32)
    ones = jnp.ones((z.shape[0], 1), jnp.float32)
    zaug_ref[:, :FF] = z[:, :FF]
    zaug_ref[:, FF:FF + 1] = ones
    zaug_ref[:, FF + 1:] = jnp.zeros((z.shape[0], WW - FF - 1), jnp.float32)
    zc = z[:, :FF]
    al = jnp.sum(zc * attn2_ref[0:1, :], axis=1)
    ar = jnp.sum(zc * attn2_ref[1:2, :], axis=1)
    alr_ref[0:1, :] = al[None, :]
    alr_ref[1:2, :] = ar[None, :]
    alr_ref[2:, :] = jnp.zeros((6, z.shape[0]), jnp.float32)
    q = jnp.dot(attne_ref[0:1, :], ew_ref[...], preferred_element_type=jnp.float32)
    we = jnp.dot(q, wedge_ref[...], preferred_element_type=jnp.float32)
    wbaug_ref[:FF, :] = wedge_ref[...]
    wbaug_ref[FF:FF + 1, :] = we
    wbaug_ref[FF + 1:, :] = jnp.zeros((WW - FF - 1, 16), jnp.float32)


def _k2_body(ef_ref, wbaug_ref, cb_ref, zeb_ref):
    zeb_ref[...] = jnp.dot(ef_ref[...], wbaug_ref[...].T,
                           preferred_element_type=jnp.float32) + cb_ref[...]


def _k4_body(hraw_ref, out_ref):
    p_lo = hraw_ref[0, 0, :NPH, :] + hraw_ref[1, 0, :NPH, :]
    p_hi = hraw_ref[0, 1, :NN - NPH, :] + hraw_ref[1, 1, :NN - NPH, :]
    p = jnp.concatenate([p_lo, p_hi], axis=0)
    d = p[:, FF:FF + 1]
    safe = jnp.where(d > 0.0, d, 1.0)
    out_ref[...] = jnp.where(d > 0.0, p[:, :FF] / safe, 0.0)


def _k3_body(zaug_hbm, alr_hbm, src_hbm, dst_hbm, zeb_hbm,
             fused_hbm, hraw_hbm, exs_hbm,
             al_t, ar_t,
             srcA, dstA, rowsA, zebA, exA,
             srcB, dstB, rowsB, zebB, exB,
             fbuf, dstx_t, zbuf, hacc,
             semrA, semzA, semrB, semzB):
    c = lax.axis_index("c")
    s = lax.axis_index("s")
    wid = c * NS + s
    ebase = wid * EPW

    lane = jnp.arange(16, dtype=jnp.int32)
    col_ee = jnp.full((16,), FF, jnp.int32)

    A = (srcA, dstA, rowsA, zebA, exA, semrA, semzA)
    B = (srcB, dstB, rowsB, zebB, exB, semrB, semzB)

    def _zero_own_slice():
        @pl.loop(0, RPS, step=ZCH)
        def _zero_hacc(r):
            pltpu.sync_copy(zbuf, hacc.at[pl.ds(s * RPS + r, ZCH)])

    @pl.loop(0, ZCH)
    def _zero_zbuf(i):
        for k in range(WW // 16):
            zbuf[i, pl.ds(k * 16, 16)] = jnp.zeros((16,), jnp.float32)

    _zero_own_slice()

    pltpu.sync_copy(alr_hbm.at[0], al_t)
    pltpu.sync_copy(alr_hbm.at[1], ar_t)
    plsc.subcore_barrier()

    def issue_idx(off, S):
        src_t, dst_t, semr = S[0], S[1], S[5]
        base = ebase + off
        pltpu.async_copy(src_hbm.at[pl.ds(base, CB)], src_t, semr)
        pltpu.async_copy(dst_hbm.at[pl.ds(base, CB)], dst_t, semr)

    def wait_idx(off, S):
        src_t, dst_t, semr = S[0], S[1], S[5]
        base = ebase + off
        pltpu.make_async_copy(src_hbm.at[pl.ds(base, CB)], src_t, semr).wait()
        pltpu.make_async_copy(dst_hbm.at[pl.ds(base, CB)], dst_t, semr).wait()

    def issue_big(off, S, phase1):
        src_t, rows_t, zeb_t, ex_t = S[0], S[2], S[3], S[4]
        semr, semz = S[5], S[6]
        base = ebase + off
        pltpu.async_copy(zaug_hbm.at[src_t], rows_t, semr)
        if phase1:
            pltpu.async_copy(zeb_hbm.at[pl.ds(base, CB)], zeb_t, semz)
        else:
            pltpu.async_copy(exs_hbm.at[pl.ds(base, CB)], ex_t.at[pl.ds(0, CB)],
                             semz)

    def compute1(off, S, prefetch):
        src_t, dst_t, rows_t, zeb_t, ex_t = S[:5]
        semr, semz = S[5], S[6]
        base = ebase + off
        pltpu.make_async_copy(zeb_hbm.at[pl.ds(base, CB)], zeb_t, semz).wait()

        for g in range(CB // 16):
            sl = pl.ds(g * 16, 16)
            si = src_t[sl]
            di = dst_t[sl]
            av = plsc.load_gather(al_t, [si])
            bv = plsc.load_gather(ar_t, [di])
            ev = plsc.load_gather(zeb_t, [lane + g * 16, col_ee])
            x = av + bv + ev
            x = jnp.where(x >= 0.0, x, x * 0.01)
            ex_t[sl] = jnp.exp(x)
            trash = NPH + ((lane + g * 16) & (NTR - 1))
            dstx_t[sl] = jnp.where(di < NPH, di, trash)

        pltpu.make_async_copy(zaug_hbm.at[src_t], rows_t, semr).wait()
        prefetch()

        @pl.loop(0, CB)
        def _edge(i):
            exs = ex_t[pl.ds(i, 16)][0]
            for k in range(WW // 16):
                ksl = pl.ds(k * 16, 16)
                r = rows_t[i, ksl]
                if k < FF // 16:
                    fbuf[i, ksl] = r + zeb_t[i, ksl]
                rows_t[i, ksl] = r * exs

        pltpu.sync_copy(fbuf, fused_hbm.at[pl.ds(base, CB)])
        pltpu.sync_copy(ex_t.at[pl.ds(0, CB)], exs_hbm.at[pl.ds(base, CB)])
        pltpu.sync_copy(rows_t, hacc.at[dstx_t], add=True)

    issue_idx(0, A)
    wait_idx(0, A)
    issue_big(0, A, True)
    issue_idx(CB, B)

    @pl.loop(0, (EPW // CB - 1) // 2 * 2 * CB, step=2 * CB)
    def _pair(off):
        wait_idx(off + CB, B)
        issue_big(off + CB, B, True)

        def _pfA():
            issue_idx(off + 2 * CB, A)
        compute1(off, A, _pfA)

        wait_idx(off + 2 * CB, A)
        issue_big(off + 2 * CB, A, True)

        def _pfB():
            @pl.when(off + 3 * CB < EPW)
            def _():
                issue_idx(off + 3 * CB, B)
        compute1(off + CB, B, _pfB)

    compute1(EPW - CB, A, lambda: None)

    plsc.subcore_barrier()
    pltpu.sync_copy(hacc.at[pl.ds(s * RPS, RPS)],
                    hraw_hbm.at[c, 0, pl.ds(s * RPS, RPS)])
    _zero_own_slice()
    plsc.subcore_barrier()

    def compute2(off, S, prefetch):
        src_t, dst_t, rows_t, zeb_t, ex_t = S[:5]
        semr, semz = S[5], S[6]
        base = ebase + off
        pltpu.make_async_copy(exs_hbm.at[pl.ds(base, CB)],
                              ex_t.at[pl.ds(0, CB)], semz).wait()

        for g in range(CB // 16):
            sl = pl.ds(g * 16, 16)
            di = dst_t[sl]
            trash = NPH + ((lane + g * 16) & (NTR - 1))
            dstx_t[sl] = jnp.where(di >= NPH, di - NPH, trash)

        pltpu.make_async_copy(zaug_hbm.at[src_t], rows_t, semr).wait()
        prefetch()

        @pl.loop(0, CB)
        def _edge2(i):
            exs = ex_t[pl.ds(i, 16)][0]
            for k in range(WW // 16):
                ksl = pl.ds(k * 16, 16)
                rows_t[i, ksl] = rows_t[i, ksl] * exs

        pltpu.sync_copy(rows_t, hacc.at[dstx_t], add=True)

    issue_idx(0, A)
    wait_idx(0, A)
    issue_big(0, A, False)
    issue_idx(CB, B)

    @pl.loop(0, (EPW // CB - 1) // 2 * 2 * CB, step=2 * CB)
    def _pair2(off):
        wait_idx(off + CB, B)
        issue_big(off + CB, B, False)

        def _pfA():
            issue_idx(off + 2 * CB, A)
        compute2(off, A, _pfA)

        wait_idx(off + 2 * CB, A)
        issue_big(off + 2 * CB, A, False)

        def _pfB():
            @pl.when(off + 3 * CB < EPW)
            def _():
                issue_idx(off + 3 * CB, B)
        compute2(off + CB, B, _pfB)

    compute2(EPW - CB, A, lambda: None)

    plsc.subcore_barrier()
    pltpu.sync_copy(hacc.at[pl.ds(s * RPS, RPS)],
                    hraw_hbm.at[c, 1, pl.ds(s * RPS, RPS)])


@functools.cache
def _edge_sc():
  return pl.kernel(
    _k3_body,
    out_type=[jax.ShapeDtypeStruct((NE, FF), jnp.float32),
              jax.ShapeDtypeStruct((NC, 2, TBL, WW), jnp.float32),
              jax.ShapeDtypeStruct((NE,), jnp.float32)],
    mesh=plsc.VectorSubcoreMesh(core_axis_name="c", subcore_axis_name="s",
                                num_cores=NC, num_subcores=NS),
    compiler_params=pltpu.CompilerParams(needs_layout_passes=False,
                                         use_tc_tiling_on_sc=False),
    scratch_types=(
        [pltpu.VMEM((NN,), jnp.float32),
         pltpu.VMEM((NN,), jnp.float32)]
        + 2 * [pltpu.VMEM((CB,), jnp.int32),
               pltpu.VMEM((CB,), jnp.int32),
               pltpu.VMEM((CB, WW), jnp.float32),
               pltpu.VMEM((CB, WW), jnp.float32),
               pltpu.VMEM((CB + 16,), jnp.float32)]
        + [pltpu.VMEM((CB, FF), jnp.float32),
           pltpu.VMEM((CB,), jnp.int32),
           pltpu.VMEM((ZCH, WW), jnp.float32),
           pltpu.VMEM_SHARED((TBL, WW), jnp.float32)]
        + 4 * [pltpu.SemaphoreType.DMA]
    ),
  )


@jax.jit
def _run(nf, src, dst, ef, W_node, W_edge, attn_l, attn_r, attn_e, edge_weights, bias):
    waug = jnp.concatenate([W_node, jnp.zeros((WW - FF, W_node.shape[1]), jnp.float32)], axis=0)
    attn2 = jnp.concatenate([attn_l, attn_r, jnp.zeros((6, FF), jnp.float32)], axis=0)
    attne8 = jnp.concatenate([attn_e, jnp.zeros((7, FF), jnp.float32)], axis=0)
    cbv = jnp.concatenate([bias, jnp.zeros((WW - FF,), jnp.float32)])[None, :]

    zaug, alr, wbaug = pl.pallas_call(
        _k1_body,
        out_shape=[jax.ShapeDtypeStruct((NN, WW), jnp.float32),
                   jax.ShapeDtypeStruct((8, NN), jnp.float32),
                   jax.ShapeDtypeStruct((WW, 16), jnp.float32)],
    )(nf, waug, attn2, W_edge, edge_weights, attne8)

    zeb = pl.pallas_call(
        _k2_body,
        grid=(NE // BE,),
        in_specs=[pl.BlockSpec((BE, 16), lambda i: (i, 0)),
                  pl.BlockSpec((WW, 16), lambda i: (0, 0)),
                  pl.BlockSpec((1, WW), lambda i: (0, 0))],
        out_specs=pl.BlockSpec((BE, WW), lambda i: (i, 0)),
        out_shape=jax.ShapeDtypeStruct((NE, WW), jnp.float32),
    )(ef, wbaug, cbv)

    fused, hraw, _ = _edge_sc()(zaug, alr, src, dst, zeb)

    n_out = pl.pallas_call(
        _k4_body,
        out_shape=jax.ShapeDtypeStruct((NN, FF), jnp.float32),
    )(hraw)
    return n_out, fused


def kernel(nf, edge_index, ef, W_node, W_edge, attn_l, attn_r, attn_e, edge_weights, bias):
    src = edge_index[0]
    dst = edge_index[1]
    n_out, fused = _run(nf, src, dst, ef.reshape(-1, 16), W_node, W_edge,
                        attn_l, attn_r, attn_e, edge_weights, bias)
    return n_out, fused.reshape(NE, 1, FF)


# speedup vs baseline: 5.7424x; 1.0101x over previous
import functools
import jax
import jax.numpy as jnp
from jax import lax
from jax.experimental import pallas as pl
from jax.experimental.pallas import tpu as pltpu
from jax.experimental.pallas import tpu_sc as plsc

NN = 10000
NE = 320000
FF = 128
WW = 144
NC = 2
NS = 16
NWK = NC * NS
EPW = NE // NWK
CB = 80
NPH = 5120
NTR = 128
TBL = NPH + NTR
RPS = TBL // NS
ZCH = 8
BE = 6400


def _k1_body(nf_ref, waug_ref, attn2_ref, wedge_ref, ew_ref, attne_ref,
             zaug_ref, alr_ref, wbaug_ref):
    z = jnp.dot(nf_ref[...], waug_ref[...].T, preferred_element_type=jnp.float32)
    ones = jnp.ones((z.shape[0], 1), jnp.float32)
    zaug_ref[:, :FF] = z[:, :FF]
    zaug_ref[:, FF:FF + 1] = ones
    zaug_ref[:, FF + 1:] = jnp.zeros((z.shape[0], WW - FF - 1), jnp.float32)
    zc = z[:, :FF]
    al = jnp.sum(zc * attn2_ref[0:1, :], axis=1)
    ar = jnp.sum(zc * attn2_ref[1:2, :], axis=1)
    alr_ref[0:1, :] = al[None, :]
    alr_ref[1:2, :] = ar[None, :]
    alr_ref[2:, :] = jnp.zeros((6, z.shape[0]), jnp.float32)
    q = jnp.dot(attne_ref[0:1, :], ew_ref[...], preferred_element_type=jnp.float32)
    we = jnp.dot(q, wedge_ref[...], preferred_element_type=jnp.float32)
    wbaug_ref[:FF, :] = wedge_ref[...]
    wbaug_ref[FF:FF + 1, :] = we
    wbaug_ref[FF + 1:, :] = jnp.zeros((WW - FF - 1, 16), jnp.float32)


def _k2_body(ef_ref, wbaug_ref, cb_ref, zeb_ref):
    zeb_ref[...] = jnp.dot(ef_ref[...], wbaug_ref[...].T,
                           preferred_element_type=jnp.float32) + cb_ref[...]


def _k4_body(hraw_ref, out_ref):
    p_lo = hraw_ref[0, 0, :NPH, :] + hraw_ref[1, 0, :NPH, :]
    p_hi = hraw_ref[0, 1, :NN - NPH, :] + hraw_ref[1, 1, :NN - NPH, :]
    p = jnp.concatenate([p_lo, p_hi], axis=0)
    d = p[:, FF:FF + 1]
    safe = jnp.where(d > 0.0, d, 1.0)
    out_ref[...] = jnp.where(d > 0.0, p[:, :FF] / safe, 0.0)


def _k3_body(zaug_hbm, alr_hbm, src_hbm, dst_hbm, zeb_hbm,
             fused_hbm, hraw_hbm, exs_hbm,
             al_t, ar_t,
             srcA, dstA, rowsA, zebA, exA,
             srcB, dstB, rowsB, zebB, exB,
             fbuf, dstx_t, zbuf, hacc,
             semrA, semzA, semrB, semzB):
    c = lax.axis_index("c")
    s = lax.axis_index("s")
    wid = c * NS + s
    ebase = wid * EPW

    lane = jnp.arange(16, dtype=jnp.int32)
    col_ee = jnp.full((16,), FF, jnp.int32)

    A = (srcA, dstA, rowsA, zebA, exA, semrA, semzA)
    B = (srcB, dstB, rowsB, zebB, exB, semrB, semzB)

    def _zero_own_slice():
        @pl.loop(0, RPS, step=ZCH)
        def _zero_hacc(r):
            pltpu.sync_copy(zbuf, hacc.at[pl.ds(s * RPS + r, ZCH)])

    @pl.loop(0, ZCH)
    def _zero_zbuf(i):
        for k in range(WW // 16):
            zbuf[i, pl.ds(k * 16, 16)] = jnp.zeros((16,), jnp.float32)

    _zero_own_slice()

    pltpu.sync_copy(alr_hbm.at[0], al_t)
    pltpu.sync_copy(alr_hbm.at[1], ar_t)
    plsc.subcore_barrier()

    def issue_idx(off, S):
        src_t, dst_t, semr = S[0], S[1], S[5]
        base = ebase + off
        pltpu.async_copy(src_hbm.at[pl.ds(base, CB)], src_t, semr)
        pltpu.async_copy(dst_hbm.at[pl.ds(base, CB)], dst_t, semr)

    def wait_idx(off, S):
        src_t, dst_t, semr = S[0], S[1], S[5]
        base = ebase + off
        pltpu.make_async_copy(src_hbm.at[pl.ds(base, CB)], src_t, semr).wait()
        pltpu.make_async_copy(dst_hbm.at[pl.ds(base, CB)], dst_t, semr).wait()

    def issue_big(off, S, phase1):
        src_t, rows_t, zeb_t, ex_t = S[0], S[2], S[3], S[4]
        semr, semz = S[5], S[6]
        base = ebase + off
        pltpu.async_copy(zaug_hbm.at[src_t], rows_t, semr)
        if phase1:
            pltpu.async_copy(zeb_hbm.at[pl.ds(base, CB)], zeb_t, semz)
        else:
            pltpu.async_copy(exs_hbm.at[pl.ds(base, CB)], ex_t.at[pl.ds(0, CB)],
                             semz)

    def compute1(off, S, prefetch):
        src_t, dst_t, rows_t, zeb_t, ex_t = S[:5]
        semr, semz = S[5], S[6]
        base = ebase + off
        pltpu.make_async_copy(zeb_hbm.at[pl.ds(base, CB)], zeb_t, semz).wait()

        for g in range(CB // 16):
            sl = pl.ds(g * 16, 16)
            si = src_t[sl]
            di = dst_t[sl]
            av = plsc.load_gather(al_t, [si])
            bv = plsc.load_gather(ar_t, [di])
            ev = plsc.load_gather(zeb_t, [lane + g * 16, col_ee])
            x = av + bv + ev
            x = jnp.where(x >= 0.0, x, x * 0.01)
            ex_t[sl] = jnp.exp(x)
            trash = NPH + ((lane + g * 16) & (NTR - 1))
            dstx_t[sl] = jnp.where(di < NPH, di, trash)

        pltpu.make_async_copy(zaug_hbm.at[src_t], rows_t, semr).wait()
        prefetch()

        @pl.loop(0, CB)
        def _edge(i):
            exs = ex_t[pl.ds(i, 16)][0]
            for k in range(WW // 16):
                ksl = pl.ds(k * 16, 16)
                r = rows_t[i, ksl]
                if k < FF // 16:
                    fbuf[i, ksl] = r + zeb_t[i, ksl]
                rows_t[i, ksl] = r * exs

        pltpu.sync_copy(fbuf, fused_hbm.at[pl.ds(base, CB)])
        pltpu.sync_copy(ex_t.at[pl.ds(0, CB)], exs_hbm.at[pl.ds(base, CB)])
        pltpu.sync_copy(rows_t, hacc.at[dstx_t], add=True)

    issue_idx(0, A)
    wait_idx(0, A)
    issue_big(0, A, True)
    issue_idx(CB, B)

    @pl.loop(0, (EPW // CB - 1) // 2 * 2 * CB, step=2 * CB)
    def _pair(off):
        wait_idx(off + CB, B)
        issue_big(off + CB, B, True)

        def _pfA():
            issue_idx(off + 2 * CB, A)
        compute1(off, A, _pfA)

        wait_idx(off + 2 * CB, A)
        issue_big(off + 2 * CB, A, True)

        def _pfB():
            @pl.when(off + 3 * CB < EPW)
            def _():
                issue_idx(off + 3 * CB, B)
        compute1(off + CB, B, _pfB)

    compute1(EPW - CB, A, lambda: None)

    plsc.subcore_barrier()
    pltpu.sync_copy(hacc.at[pl.ds(s * RPS, RPS)],
                    hraw_hbm.at[c, 0, pl.ds(s * RPS, RPS)])
    _zero_own_slice()
    plsc.subcore_barrier()

    def compute2(off, S, prefetch):
        src_t, dst_t, rows_t, zeb_t, ex_t = S[:5]
        semr, semz = S[5], S[6]
        base = ebase + off
        pltpu.make_async_copy(exs_hbm.at[pl.ds(base, CB)],
                              ex_t.at[pl.ds(0, CB)], semz).wait()

        for g in range(CB // 16):
            sl = pl.ds(g * 16, 16)
            di = dst_t[sl]
            trash = NPH + ((lane + g * 16) & (NTR - 1))
            dstx_t[sl] = jnp.where(di >= NPH, di - NPH, trash)

        pltpu.make_async_copy(zaug_hbm.at[src_t], rows_t, semr).wait()
        prefetch()

        @pl.loop(0, CB)
        def _edge2(i):
            exs = ex_t[pl.ds(i, 16)][0]
            for k in range(WW // 16):
                ksl = pl.ds(k * 16, 16)
                rows_t[i, ksl] = rows_t[i, ksl] * exs

        pltpu.sync_copy(rows_t, hacc.at[dstx_t], add=True)

    issue_idx(0, A)
    wait_idx(0, A)
    issue_big(0, A, False)
    issue_idx(CB, B)

    @pl.loop(0, (EPW // CB - 1) // 2 * 2 * CB, step=2 * CB)
    def _pair2(off):
        wait_idx(off + CB, B)
        issue_big(off + CB, B, False)

        def _pfA():
            issue_idx(off + 2 * CB, A)
        compute2(off, A, _pfA)

        wait_idx(off + 2 * CB, A)
        issue_big(off + 2 * CB, A, False)

        def _pfB():
            @pl.when(off + 3 * CB < EPW)
            def _():
                issue_idx(off + 3 * CB, B)
        compute2(off + CB, B, _pfB)

    compute2(EPW - CB, A, lambda: None)

    plsc.subcore_barrier()
    pltpu.sync_copy(hacc.at[pl.ds(s * RPS, RPS)],
                    hraw_hbm.at[c, 1, pl.ds(s * RPS, RPS)])


@functools.cache
def _edge_sc():
  return pl.kernel(
    _k3_body,
    out_type=[jax.ShapeDtypeStruct((NE, FF), jnp.float32),
              jax.ShapeDtypeStruct((NC, 2, TBL, WW), jnp.float32),
              jax.ShapeDtypeStruct((NE,), jnp.float32)],
    mesh=plsc.VectorSubcoreMesh(core_axis_name="c", subcore_axis_name="s",
                                num_cores=NC, num_subcores=NS),
    compiler_params=pltpu.CompilerParams(needs_layout_passes=False,
                                         use_tc_tiling_on_sc=False),
    scratch_types=(
        [pltpu.VMEM((NN,), jnp.float32),
         pltpu.VMEM((NN,), jnp.float32)]
        + 2 * [pltpu.VMEM((CB,), jnp.int32),
               pltpu.VMEM((CB,), jnp.int32),
               pltpu.VMEM((CB, WW), jnp.float32),
               pltpu.VMEM((CB, WW), jnp.float32),
               pltpu.VMEM((CB + 16,), jnp.float32)]
        + [pltpu.VMEM((CB, FF), jnp.float32),
           pltpu.VMEM((CB,), jnp.int32),
           pltpu.VMEM((ZCH, WW), jnp.float32),
           pltpu.VMEM_SHARED((TBL, WW), jnp.float32)]
        + 4 * [pltpu.SemaphoreType.DMA]
    ),
  )


@jax.jit
def _run(nf, src, dst, ef, W_node, W_edge, attn_l, attn_r, attn_e, edge_weights, bias):
    waug = jnp.concatenate([W_node, jnp.zeros((WW - FF, W_node.shape[1]), jnp.float32)], axis=0)
    attn2 = jnp.concatenate([attn_l, attn_r, jnp.zeros((6, FF), jnp.float32)], axis=0)
    attne8 = jnp.concatenate([attn_e, jnp.zeros((7, FF), jnp.float32)], axis=0)
    cbv = jnp.concatenate([bias, jnp.zeros((WW - FF,), jnp.float32)])[None, :]

    zaug, alr, wbaug = pl.pallas_call(
        _k1_body,
        out_shape=[jax.ShapeDtypeStruct((NN, WW), jnp.float32),
                   jax.ShapeDtypeStruct((8, NN), jnp.float32),
                   jax.ShapeDtypeStruct((WW, 16), jnp.float32)],
    )(nf, waug, attn2, W_edge, edge_weights, attne8)

    zeb = pl.pallas_call(
        _k2_body,
        grid=(NE // BE,),
        in_specs=[pl.BlockSpec((BE, 16), lambda i: (i, 0)),
                  pl.BlockSpec((WW, 16), lambda i: (0, 0)),
                  pl.BlockSpec((1, WW), lambda i: (0, 0))],
        out_specs=pl.BlockSpec((BE, WW), lambda i: (i, 0)),
        out_shape=jax.ShapeDtypeStruct((NE, WW), jnp.float32),
    )(ef, wbaug, cbv)

    fused, hraw, _ = _edge_sc()(zaug, alr, src, dst, zeb)

    n_out = pl.pallas_call(
        _k4_body,
        out_shape=jax.ShapeDtypeStruct((NN, FF), jnp.float32),
    )(hraw)
    return n_out, fused


def kernel(nf, edge_index, ef, W_node, W_edge, attn_l, attn_r, attn_e, edge_weights, bias):
    src = edge_index[0]
    dst = edge_index[1]
    n_out, fused = _run(nf, src, dst, ef.reshape(-1, 16), W_node, W_edge,
                        attn_l, attn_r, attn_e, edge_weights, bias)
    return n_out, fused.reshape(NE, 1, FF)


# speedup vs baseline: 5.7576x; 1.0026x over previous
import functools
import jax
import jax.numpy as jnp
from jax import lax
from jax.experimental import pallas as pl
from jax.experimental.pallas import tpu as pltpu
from jax.experimental.pallas import tpu_sc as plsc

NN = 10000
NE = 320000
FF = 128
WW = 144
NC = 2
NS = 16
NWK = NC * NS
EPW = NE // NWK
CB = 80
NPH = 5120
NTR = 128
TBL = NPH + NTR
RPS = TBL // NS
ZCH = 8
BE = 6400


def _k1_body(nf_ref, waug_ref, attn2_ref, wedge_ref, ew_ref, attne_ref,
             zaug_ref, alr_ref, wbaug_ref):
    z = jnp.dot(nf_ref[...], waug_ref[...].T, preferred_element_type=jnp.float32)
    ones = jnp.ones((z.shape[0], 1), jnp.float32)
    zaug_ref[:, :FF] = z[:, :FF]
    zaug_ref[:, FF:FF + 1] = ones
    zaug_ref[:, FF + 1:] = jnp.zeros((z.shape[0], WW - FF - 1), jnp.float32)
    zc = z[:, :FF]
    al = jnp.sum(zc * attn2_ref[0:1, :], axis=1)
    ar = jnp.sum(zc * attn2_ref[1:2, :], axis=1)
    alr_ref[0:1, :] = al[None, :]
    alr_ref[1:2, :] = ar[None, :]
    alr_ref[2:, :] = jnp.zeros((6, z.shape[0]), jnp.float32)
    q = jnp.dot(attne_ref[0:1, :], ew_ref[...], preferred_element_type=jnp.float32)
    we = jnp.dot(q, wedge_ref[...], preferred_element_type=jnp.float32)
    wbaug_ref[:FF, :] = wedge_ref[...]
    wbaug_ref[FF:FF + 1, :] = we
    wbaug_ref[FF + 1:, :] = jnp.zeros((WW - FF - 1, 16), jnp.float32)


def _k2_body(ef_ref, wbaug_ref, cb_ref, zeb_ref):
    zeb_ref[...] = jnp.dot(ef_ref[...], wbaug_ref[...].T,
                           preferred_element_type=jnp.float32) + cb_ref[...]


def _k4_body(hraw_ref, out_ref):
    p_lo = hraw_ref[0, 0, :NPH, :] + hraw_ref[1, 0, :NPH, :]
    p_hi = hraw_ref[0, 1, :NN - NPH, :] + hraw_ref[1, 1, :NN - NPH, :]
    p = jnp.concatenate([p_lo, p_hi], axis=0)
    d = p[:, FF:FF + 1]
    safe = jnp.where(d > 0.0, d, 1.0)
    out_ref[...] = jnp.where(d > 0.0, p[:, :FF] / safe, 0.0)


def _k3_body(zaug_hbm, alr_hbm, src_hbm, dst_hbm, zeb_hbm,
             fused_hbm, hraw_hbm, exs_hbm,
             al_t, ar_t,
             srcA, dstA, rowsA, zebA, exA,
             srcB, dstB, rowsB, zebB, exB,
             fbuf, dstx_t, zbuf, hacc,
             semrA, semzA, semrB, semzB):
    c = lax.axis_index("c")
    s = lax.axis_index("s")
    wid = c * NS + s
    ebase = wid * EPW

    lane = jnp.arange(16, dtype=jnp.int32)
    col_ee = jnp.full((16,), FF, jnp.int32)

    A = (srcA, dstA, rowsA, zebA, exA, semrA, semzA)
    B = (srcB, dstB, rowsB, zebB, exB, semrB, semzB)

    def _zero_own_slice():
        @pl.loop(0, CB)
        def _zero_rows(i):
            for k in range(WW // 16):
                rowsA[i, pl.ds(k * 16, 16)] = jnp.zeros((16,), jnp.float32)

        @pl.loop(0, RPS - ZCH, step=CB)
        def _zero_hacc(r):
            pltpu.sync_copy(rowsA, hacc.at[pl.ds(s * RPS + r, CB)])

        pltpu.sync_copy(zbuf, hacc.at[pl.ds(s * RPS + RPS - ZCH, ZCH)])

    @pl.loop(0, ZCH)
    def _zero_zbuf(i):
        for k in range(WW // 16):
            zbuf[i, pl.ds(k * 16, 16)] = jnp.zeros((16,), jnp.float32)

    _zero_own_slice()

    pltpu.sync_copy(alr_hbm.at[0], al_t)
    pltpu.sync_copy(alr_hbm.at[1], ar_t)
    plsc.subcore_barrier()

    def issue_idx(off, S):
        src_t, dst_t, semr = S[0], S[1], S[5]
        base = ebase + off
        pltpu.async_copy(src_hbm.at[pl.ds(base, CB)], src_t, semr)
        pltpu.async_copy(dst_hbm.at[pl.ds(base, CB)], dst_t, semr)

    def wait_idx(off, S):
        src_t, dst_t, semr = S[0], S[1], S[5]
        base = ebase + off
        pltpu.make_async_copy(src_hbm.at[pl.ds(base, CB)], src_t, semr).wait()
        pltpu.make_async_copy(dst_hbm.at[pl.ds(base, CB)], dst_t, semr).wait()

    def issue_big(off, S, phase1):
        src_t, rows_t, zeb_t, ex_t = S[0], S[2], S[3], S[4]
        semr, semz = S[5], S[6]
        base = ebase + off
        pltpu.async_copy(zaug_hbm.at[src_t], rows_t, semr)
        if phase1:
            pltpu.async_copy(zeb_hbm.at[pl.ds(base, CB)], zeb_t, semz)
        else:
            pltpu.async_copy(exs_hbm.at[pl.ds(base, CB)], ex_t.at[pl.ds(0, CB)],
                             semz)

    def compute1(off, S, prefetch):
        src_t, dst_t, rows_t, zeb_t, ex_t = S[:5]
        semr, semz = S[5], S[6]
        base = ebase + off
        pltpu.make_async_copy(zeb_hbm.at[pl.ds(base, CB)], zeb_t, semz).wait()

        for g in range(CB // 16):
            sl = pl.ds(g * 16, 16)
            si = src_t[sl]
            di = dst_t[sl]
            av = plsc.load_gather(al_t, [si])
            bv = plsc.load_gather(ar_t, [di])
            ev = plsc.load_gather(zeb_t, [lane + g * 16, col_ee])
            x = av + bv + ev
            x = jnp.where(x >= 0.0, x, x * 0.01)
            ex_t[sl] = jnp.exp(x)
            trash = NPH + ((lane + g * 16) & (NTR - 1))
            dstx_t[sl] = jnp.where(di < NPH, di, trash)

        pltpu.make_async_copy(zaug_hbm.at[src_t], rows_t, semr).wait()
        prefetch()

        @pl.loop(0, CB)
        def _edge(i):
            exs = ex_t[pl.ds(i, 16)][0]
            for k in range(WW // 16):
                ksl = pl.ds(k * 16, 16)
                r = rows_t[i, ksl]
                if k < FF // 16:
                    fbuf[i, ksl] = r + zeb_t[i, ksl]
                rows_t[i, ksl] = r * exs

        pltpu.sync_copy(fbuf, fused_hbm.at[pl.ds(base, CB)])
        pltpu.sync_copy(ex_t.at[pl.ds(0, CB)], exs_hbm.at[pl.ds(base, CB)])
        pltpu.sync_copy(rows_t, hacc.at[dstx_t], add=True)

    issue_idx(0, A)
    wait_idx(0, A)
    issue_big(0, A, True)
    issue_idx(CB, B)

    @pl.loop(0, (EPW // CB - 1) // 2 * 2 * CB, step=2 * CB)
    def _pair(off):
        wait_idx(off + CB, B)
        issue_big(off + CB, B, True)

        def _pfA():
            issue_idx(off + 2 * CB, A)
        compute1(off, A, _pfA)

        wait_idx(off + 2 * CB, A)
        issue_big(off + 2 * CB, A, True)

        def _pfB():
            @pl.when(off + 3 * CB < EPW)
            def _():
                issue_idx(off + 3 * CB, B)
        compute1(off + CB, B, _pfB)

    compute1(EPW - CB, A, lambda: None)

    plsc.subcore_barrier()
    pltpu.sync_copy(hacc.at[pl.ds(s * RPS, RPS)],
                    hraw_hbm.at[c, 0, pl.ds(s * RPS, RPS)])
    _zero_own_slice()
    plsc.subcore_barrier()

    def compute2(off, S, prefetch):
        src_t, dst_t, rows_t, zeb_t, ex_t = S[:5]
        semr, semz = S[5], S[6]
        base = ebase + off
        pltpu.make_async_copy(exs_hbm.at[pl.ds(base, CB)],
                              ex_t.at[pl.ds(0, CB)], semz).wait()

        for g in range(CB // 16):
            sl = pl.ds(g * 16, 16)
            di = dst_t[sl]
            trash = NPH + ((lane + g * 16) & (NTR - 1))
            dstx_t[sl] = jnp.where(di >= NPH, di - NPH, trash)

        pltpu.make_async_copy(zaug_hbm.at[src_t], rows_t, semr).wait()
        prefetch()

        @pl.loop(0, CB)
        def _edge2(i):
            exs = ex_t[pl.ds(i, 16)][0]
            for k in range(WW // 16):
                ksl = pl.ds(k * 16, 16)
                rows_t[i, ksl] = rows_t[i, ksl] * exs

        pltpu.sync_copy(rows_t, hacc.at[dstx_t], add=True)

    issue_idx(0, A)
    wait_idx(0, A)
    issue_big(0, A, False)
    issue_idx(CB, B)

    @pl.loop(0, (EPW // CB - 1) // 2 * 2 * CB, step=2 * CB)
    def _pair2(off):
        wait_idx(off + CB, B)
        issue_big(off + CB, B, False)

        def _pfA():
            issue_idx(off + 2 * CB, A)
        compute2(off, A, _pfA)

        wait_idx(off + 2 * CB, A)
        issue_big(off + 2 * CB, A, False)

        def _pfB():
            @pl.when(off + 3 * CB < EPW)
            def _():
                issue_idx(off + 3 * CB, B)
        compute2(off + CB, B, _pfB)

    compute2(EPW - CB, A, lambda: None)

    plsc.subcore_barrier()
    pltpu.sync_copy(hacc.at[pl.ds(s * RPS, RPS)],
                    hraw_hbm.at[c, 1, pl.ds(s * RPS, RPS)])


@functools.cache
def _edge_sc():
  return pl.kernel(
    _k3_body,
    out_type=[jax.ShapeDtypeStruct((NE, FF), jnp.float32),
              jax.ShapeDtypeStruct((NC, 2, TBL, WW), jnp.float32),
              jax.ShapeDtypeStruct((NE,), jnp.float32)],
    mesh=plsc.VectorSubcoreMesh(core_axis_name="c", subcore_axis_name="s",
                                num_cores=NC, num_subcores=NS),
    compiler_params=pltpu.CompilerParams(needs_layout_passes=False,
                                         use_tc_tiling_on_sc=False),
    scratch_types=(
        [pltpu.VMEM((NN,), jnp.float32),
         pltpu.VMEM((NN,), jnp.float32)]
        + 2 * [pltpu.VMEM((CB,), jnp.int32),
               pltpu.VMEM((CB,), jnp.int32),
               pltpu.VMEM((CB, WW), jnp.float32),
               pltpu.VMEM((CB, WW), jnp.float32),
               pltpu.VMEM((CB + 16,), jnp.float32)]
        + [pltpu.VMEM((CB, FF), jnp.float32),
           pltpu.VMEM((CB,), jnp.int32),
           pltpu.VMEM((ZCH, WW), jnp.float32),
           pltpu.VMEM_SHARED((TBL, WW), jnp.float32)]
        + 4 * [pltpu.SemaphoreType.DMA]
    ),
  )


@jax.jit
def _run(nf, src, dst, ef, W_node, W_edge, attn_l, attn_r, attn_e, edge_weights, bias):
    waug = jnp.concatenate([W_node, jnp.zeros((WW - FF, W_node.shape[1]), jnp.float32)], axis=0)
    attn2 = jnp.concatenate([attn_l, attn_r, jnp.zeros((6, FF), jnp.float32)], axis=0)
    attne8 = jnp.concatenate([attn_e, jnp.zeros((7, FF), jnp.float32)], axis=0)
    cbv = jnp.concatenate([bias, jnp.zeros((WW - FF,), jnp.float32)])[None, :]

    zaug, alr, wbaug = pl.pallas_call(
        _k1_body,
        out_shape=[jax.ShapeDtypeStruct((NN, WW), jnp.float32),
                   jax.ShapeDtypeStruct((8, NN), jnp.float32),
                   jax.ShapeDtypeStruct((WW, 16), jnp.float32)],
    )(nf, waug, attn2, W_edge, edge_weights, attne8)

    zeb = pl.pallas_call(
        _k2_body,
        grid=(NE // BE,),
        in_specs=[pl.BlockSpec((BE, 16), lambda i: (i, 0)),
                  pl.BlockSpec((WW, 16), lambda i: (0, 0)),
                  pl.BlockSpec((1, WW), lambda i: (0, 0))],
        out_specs=pl.BlockSpec((BE, WW), lambda i: (i, 0)),
        out_shape=jax.ShapeDtypeStruct((NE, WW), jnp.float32),
    )(ef, wbaug, cbv)

    fused, hraw, _ = _edge_sc()(zaug, alr, src, dst, zeb)

    n_out = pl.pallas_call(
        _k4_body,
        out_shape=jax.ShapeDtypeStruct((NN, FF), jnp.float32),
    )(hraw)
    return n_out, fused


def kernel(nf, edge_index, ef, W_node, W_edge, attn_l, attn_r, attn_e, edge_weights, bias):
    src = edge_index[0]
    dst = edge_index[1]
    n_out, fused = _run(nf, src, dst, ef.reshape(-1, 16), W_node, W_edge,
                        attn_l, attn_r, attn_e, edge_weights, bias)
    return n_out, fused.reshape(NE, 1, FF)
